# baseline (speedup 1.0000x reference)
import jax
import jax.numpy as jnp
from jax import lax
from jax.experimental import pallas as pl
from jax.experimental.pallas import tpu as pltpu
from jax.experimental.pallas import tpu_sc as plsc

NC = 2
NS = 16
NW = NC * NS
B = 16384
D = 16
BPW = B // NW


def _gmf_body(u_idx_hbm, i_idx_hbm, ut_hbm, it_hbm, out_hbm,
              idx_u_v, idx_i_v, rows_u_v, rows_i_v, sem_u, sem_i):
    wid = lax.axis_index("s") * NC + lax.axis_index("c")
    base = wid * BPW

    pltpu.sync_copy(u_idx_hbm.at[pl.ds(base, BPW)], idx_u_v)
    pltpu.sync_copy(i_idx_hbm.at[pl.ds(base, BPW)], idx_i_v)

    @pl.loop(0, BPW // D)
    def _(c):
        r0 = c * D
        iu_vec = idx_u_v[pl.ds(r0, D)]
        ii_vec = idx_i_v[pl.ds(r0, D)]
        for j in range(D):
            ou = pl.multiple_of(iu_vec[j] * D, D)
            oi = pl.multiple_of(ii_vec[j] * D, D)
            od = pl.multiple_of((r0 + j) * D, D)
            pltpu.async_copy(ut_hbm.at[pl.ds(ou, D)],
                             rows_u_v.at[pl.ds(od, D)], sem_u)
            pltpu.async_copy(it_hbm.at[pl.ds(oi, D)],
                             rows_i_v.at[pl.ds(od, D)], sem_i)

    pltpu.make_async_copy(ut_hbm.at[pl.ds(0, BPW * D)], rows_u_v, sem_u).wait()
    pltpu.make_async_copy(it_hbm.at[pl.ds(0, BPW * D)], rows_i_v, sem_i).wait()

    @pl.loop(0, BPW)
    def _(j):
        s = pl.ds(pl.multiple_of(j * D, D), D)
        rows_u_v[s] = rows_u_v[s] * rows_i_v[s]

    ob = pl.multiple_of(base * D, BPW * D)
    pltpu.sync_copy(rows_u_v, out_hbm.at[pl.ds(ob, BPW * D)])


def kernel(inputs, user_table, item_table):
    idx = inputs.astype(jnp.int32)
    u_idx = idx[:, 0]
    i_idx = idx[:, 1]

    run = pl.kernel(
        _gmf_body,
        out_type=jax.ShapeDtypeStruct((B * D,), jnp.float32),
        mesh=plsc.VectorSubcoreMesh(core_axis_name="c", subcore_axis_name="s"),
        scratch_types=[
            pltpu.VMEM((BPW,), jnp.int32),
            pltpu.VMEM((BPW,), jnp.int32),
            pltpu.VMEM((BPW * D,), jnp.float32),
            pltpu.VMEM((BPW * D,), jnp.float32),
            pltpu.SemaphoreType.DMA,
            pltpu.SemaphoreType.DMA,
        ],
    )
    out = run(u_idx, i_idx, user_table.reshape(-1), item_table.reshape(-1))
    return out.reshape(B, D)
